# baseline (speedup 1.0000x reference)
import functools

import jax
import jax.numpy as jnp
from jax.experimental import pallas as pl
from jax.experimental.pallas import tpu as pltpu

B, N, DNODE, DCENT, H = 1024, 64, 128, 128, 256
MLP_IN = DCENT + H
BB = 128


def _gnn_body(node_ref, adj_ref, aid_ref, We_ref, be_ref,
              Wg1_ref, bg1_ref, m2_ref):
    f32 = jnp.float32
    bf16 = jnp.bfloat16
    X = node_ref[...].reshape(BB * N, DNODE).astype(bf16)
    h0 = jnp.maximum(
        jnp.dot(X, We_ref[...].astype(bf16), preferred_element_type=f32)
        + be_ref[...], 0.0).astype(bf16)
    adjt = adj_ref[...]
    adjb = jnp.transpose(adjt, (2, 0, 1))
    deg = jnp.maximum(jnp.sum(adjb.astype(f32), axis=2, keepdims=True), 1e-6)
    A = adjb * (1.0 / deg).astype(bf16)
    g = jnp.dot(h0, Wg1_ref[...].astype(bf16),
                preferred_element_type=f32)
    g3 = g.astype(bf16).reshape(BB, N, H)
    m = jax.lax.dot_general(A, g3,
                            (((2,), (1,)), ((0,), (0,))),
                            preferred_element_type=f32)
    h1 = jnp.maximum(m + bg1_ref[...], 0.0)
    aid2 = jnp.transpose(aid_ref[...])
    nidx = jax.lax.broadcasted_iota(jnp.int32, (BB, N), 1)
    onehotf = (nidx == aid2).astype(bf16)
    arow = jax.lax.dot_general(onehotf, A, (((1,), (1,)), ((0,), (0,))),
                               preferred_element_type=f32)
    m2_ref[...] = jax.lax.dot_general(
        arow, h1, (((1,), (1,)), ((0,), (0,))),
        preferred_element_type=f32)


def _head_body(m2_ref, cent_ref, Wg2_ref, bg2_ref, gam_ref, bet_ref,
               W1_ref, b1_ref, W2_ref, b2_ref, Wv_ref, bv_ref, out_ref):
    f32 = jnp.float32
    bf16 = jnp.bfloat16
    feats = jnp.maximum(
        jnp.dot(m2_ref[...].astype(bf16), Wg2_ref[...].astype(bf16),
                preferred_element_type=f32) + bg2_ref[...], 0.0)
    inp = jnp.concatenate([cent_ref[...], feats], axis=1)
    mu = jnp.mean(inp, axis=1, keepdims=True)
    var = jnp.mean(inp * inp, axis=1, keepdims=True) - mu * mu
    x = (inp - mu) * jax.lax.rsqrt(var + 1e-5) * gam_ref[...] + bet_ref[...]
    x = jnp.maximum(jnp.dot(x.astype(bf16), W1_ref[...].astype(bf16),
                            preferred_element_type=f32) + b1_ref[...], 0.0)
    x = jnp.maximum(jnp.dot(x.astype(bf16), W2_ref[...].astype(bf16),
                            preferred_element_type=f32) + b2_ref[...], 0.0)
    out_ref[...] = jnp.sum(x * Wv_ref[...], axis=1, keepdims=True) \
        + bv_ref[...]


@functools.partial(jax.jit, static_argnames=())
def kernel(cent_obs, node_obs, adj, agent_id, W_embed, b_embed, Wg1, bg1,
           Wg2, bg2, gamma, beta, W1, b1, W2, b2, Wv, bv):
    bf16 = jnp.bfloat16
    nb = B // BB
    full = lambda shp: pl.BlockSpec(shp, lambda i: (0,) * len(shp))
    m2 = pl.pallas_call(
        _gnn_body,
        grid_spec=pl.GridSpec(
            grid=(nb,),
            in_specs=[
                pl.BlockSpec((BB, N, DNODE), lambda i: (i, 0, 0)),
                pl.BlockSpec((N, N, BB), lambda i: (0, 0, i)),
                pl.BlockSpec((1, BB), lambda i: (0, i)),
                full((DNODE, H)), full((1, H)),
                full((H, H)), full((1, H)),
            ],
            out_specs=pl.BlockSpec((BB, H), lambda i: (i, 0)),
        ),
        out_shape=jax.ShapeDtypeStruct((B, H), jnp.float32),
        compiler_params=pltpu.CompilerParams(
            dimension_semantics=("arbitrary",)),
    )(node_obs, jnp.transpose(adj.astype(jnp.bfloat16), (1, 2, 0)),
      jnp.transpose(agent_id.astype(jnp.int32)),
      W_embed, b_embed.reshape(1, H),
      Wg1, bg1.reshape(1, H))

    full1 = lambda shp: pl.BlockSpec(shp, lambda: (0,) * len(shp))
    out = pl.pallas_call(
        _head_body,
        grid_spec=pl.GridSpec(
            grid=(),
            in_specs=[
                full1((B, H)), full1((B, DCENT)),
                full1((H, H)), full1((1, H)),
                full1((1, MLP_IN)), full1((1, MLP_IN)),
                full1((MLP_IN, H)), full1((1, H)),
                full1((H, H)), full1((1, H)),
                full1((1, H)), full1((1, 1)),
            ],
            out_specs=full1((B, 1)),
        ),
        out_shape=jax.ShapeDtypeStruct((B, 1), jnp.float32),
    )(m2, cent_obs,
      Wg2, bg2.reshape(1, H),
      gamma.reshape(1, MLP_IN), beta.reshape(1, MLP_IN),
      W1, b1.reshape(1, H),
      W2, b2.reshape(1, H),
      jnp.transpose(Wv), bv.reshape(1, 1))
    return out


# speedup vs baseline: 1.1570x; 1.1570x over previous
import functools

import jax
import jax.numpy as jnp
from jax.experimental import pallas as pl
from jax.experimental.pallas import tpu as pltpu

B, N, DNODE, DCENT, H = 1024, 64, 128, 128, 256
MLP_IN = DCENT + H
BB = 128


def _gnn_body(node_ref, adj_ref, aid_ref, We_ref, be_ref,
              Wg1_ref, bg1_ref, m2_ref):
    f32 = jnp.float32
    bf16 = jnp.bfloat16
    X = node_ref[...].reshape(BB * N, DNODE).astype(bf16)
    h0 = jnp.maximum(
        jnp.dot(X, We_ref[...].astype(bf16), preferred_element_type=f32)
        + be_ref[...], 0.0).astype(bf16)
    adjt = adj_ref[...]
    adjb = jnp.transpose(adjt, (2, 0, 1))
    deg = jnp.maximum(jnp.sum(adjb, axis=2, keepdims=True), 1e-6)
    A = (adjb * (1.0 / deg)).astype(bf16)
    g = jnp.dot(h0, Wg1_ref[...].astype(bf16),
                preferred_element_type=f32)
    g3 = g.astype(bf16).reshape(BB, N, H)
    m = jax.lax.dot_general(A, g3,
                            (((2,), (1,)), ((0,), (0,))),
                            preferred_element_type=f32)
    h1 = jnp.maximum(m + bg1_ref[...], 0.0)
    aid2 = jnp.transpose(aid_ref[...])
    nidx = jax.lax.broadcasted_iota(jnp.int32, (BB, N), 1)
    onehotf = (nidx == aid2).astype(bf16)
    arow = jax.lax.dot_general(onehotf, A, (((1,), (1,)), ((0,), (0,))),
                               preferred_element_type=f32)
    m2_ref[...] = jax.lax.dot_general(
        arow, h1, (((1,), (1,)), ((0,), (0,))),
        preferred_element_type=f32)


def _head_body(m2_ref, cent_ref, Wg2_ref, bg2_ref, gam_ref, bet_ref,
               W1_ref, b1_ref, W2_ref, b2_ref, Wv_ref, bv_ref, out_ref):
    f32 = jnp.float32
    bf16 = jnp.bfloat16
    feats = jnp.maximum(
        jnp.dot(m2_ref[...].astype(bf16), Wg2_ref[...].astype(bf16),
                preferred_element_type=f32) + bg2_ref[...], 0.0)
    inp = jnp.concatenate([cent_ref[...], feats], axis=1)
    mu = jnp.mean(inp, axis=1, keepdims=True)
    var = jnp.mean(inp * inp, axis=1, keepdims=True) - mu * mu
    x = (inp - mu) * jax.lax.rsqrt(var + 1e-5) * gam_ref[...] + bet_ref[...]
    x = jnp.maximum(jnp.dot(x.astype(bf16), W1_ref[...].astype(bf16),
                            preferred_element_type=f32) + b1_ref[...], 0.0)
    x = jnp.maximum(jnp.dot(x.astype(bf16), W2_ref[...].astype(bf16),
                            preferred_element_type=f32) + b2_ref[...], 0.0)
    out_ref[...] = jnp.sum(x * Wv_ref[...], axis=1, keepdims=True) \
        + bv_ref[...]


@functools.partial(jax.jit, static_argnames=())
def kernel(cent_obs, node_obs, adj, agent_id, W_embed, b_embed, Wg1, bg1,
           Wg2, bg2, gamma, beta, W1, b1, W2, b2, Wv, bv):
    bf16 = jnp.bfloat16
    nb = B // BB
    full = lambda shp: pl.BlockSpec(shp, lambda i: (0,) * len(shp))
    m2 = pl.pallas_call(
        _gnn_body,
        grid_spec=pl.GridSpec(
            grid=(nb,),
            in_specs=[
                pl.BlockSpec((BB, N, DNODE), lambda i: (i, 0, 0)),
                pl.BlockSpec((N, N, BB), lambda i: (0, 0, i)),
                pl.BlockSpec((1, BB), lambda i: (0, i)),
                full((DNODE, H)), full((1, H)),
                full((H, H)), full((1, H)),
            ],
            out_specs=pl.BlockSpec((BB, H), lambda i: (i, 0)),
        ),
        out_shape=jax.ShapeDtypeStruct((B, H), jnp.float32),
        compiler_params=pltpu.CompilerParams(
            dimension_semantics=("arbitrary",)),
    )(node_obs, jnp.transpose(adj, (1, 2, 0)),
      jnp.transpose(agent_id.astype(jnp.int32)),
      W_embed, b_embed.reshape(1, H),
      Wg1, bg1.reshape(1, H))

    full1 = lambda shp: pl.BlockSpec(shp, lambda: (0,) * len(shp))
    out = pl.pallas_call(
        _head_body,
        grid_spec=pl.GridSpec(
            grid=(),
            in_specs=[
                full1((B, H)), full1((B, DCENT)),
                full1((H, H)), full1((1, H)),
                full1((1, MLP_IN)), full1((1, MLP_IN)),
                full1((MLP_IN, H)), full1((1, H)),
                full1((H, H)), full1((1, H)),
                full1((1, H)), full1((1, 1)),
            ],
            out_specs=full1((B, 1)),
        ),
        out_shape=jax.ShapeDtypeStruct((B, 1), jnp.float32),
    )(m2, cent_obs,
      Wg2, bg2.reshape(1, H),
      gamma.reshape(1, MLP_IN), beta.reshape(1, MLP_IN),
      W1, b1.reshape(1, H),
      W2, b2.reshape(1, H),
      jnp.transpose(Wv), bv.reshape(1, 1))
    return out


# speedup vs baseline: 1.1935x; 1.0316x over previous
import functools

import jax
import jax.numpy as jnp
from jax.experimental import pallas as pl
from jax.experimental.pallas import tpu as pltpu

B, N, DNODE, DCENT, H = 1024, 64, 128, 128, 256
MLP_IN = DCENT + H
BB = 128


def _gnn_body(node_ref, adj_ref, aid_ref, We_ref, be_ref,
              Wg1_ref, bg1_ref, m2_ref):
    f32 = jnp.float32
    bf16 = jnp.bfloat16
    X = node_ref[...].reshape(BB * N, DNODE).astype(bf16)
    h0 = jnp.maximum(
        jnp.dot(X, We_ref[...].astype(bf16), preferred_element_type=f32)
        + be_ref[...], 0.0).astype(bf16)
    adjt = adj_ref[...]
    adjb = jnp.transpose(adjt, (2, 0, 1))
    deg = jnp.maximum(jnp.sum(adjb, axis=2, keepdims=True), 1e-6)
    A = (adjb * (1.0 / deg)).astype(bf16)
    g = jnp.dot(h0, Wg1_ref[...].astype(bf16),
                preferred_element_type=f32)
    g3 = g.astype(bf16).reshape(BB, N, H)
    m = jax.lax.dot_general(A, g3,
                            (((2,), (1,)), ((0,), (0,))),
                            preferred_element_type=f32)
    h1 = jnp.maximum(m + bg1_ref[...], 0.0)
    aid2 = jnp.transpose(aid_ref[...])
    nidx = jax.lax.broadcasted_iota(jnp.int32, (BB, N), 1)
    onehotf = (nidx == aid2).astype(bf16)
    arow = jax.lax.dot_general(onehotf, A, (((1,), (1,)), ((0,), (0,))),
                               preferred_element_type=f32)
    m2_ref[...] = jax.lax.dot_general(
        arow, h1, (((1,), (1,)), ((0,), (0,))),
        preferred_element_type=f32)


def _head_body(m2_ref, cent_ref, Wg2_ref, bg2_ref, gam_ref, bet_ref,
               W1_ref, b1_ref, W2_ref, b2_ref, Wv_ref, bv_ref, out_ref):
    f32 = jnp.float32
    bf16 = jnp.bfloat16
    feats = jnp.maximum(
        jnp.dot(m2_ref[...].astype(bf16), Wg2_ref[...].astype(bf16),
                preferred_element_type=f32) + bg2_ref[...], 0.0)
    inp = jnp.concatenate([cent_ref[...], feats], axis=1)
    mu = jnp.mean(inp, axis=1, keepdims=True)
    var = jnp.mean(inp * inp, axis=1, keepdims=True) - mu * mu
    x = (inp - mu) * jax.lax.rsqrt(var + 1e-5) * gam_ref[...] + bet_ref[...]
    x = jnp.maximum(jnp.dot(x.astype(bf16), W1_ref[...].astype(bf16),
                            preferred_element_type=f32) + b1_ref[...], 0.0)
    x = jnp.maximum(jnp.dot(x.astype(bf16), W2_ref[...].astype(bf16),
                            preferred_element_type=f32) + b2_ref[...], 0.0)
    out_ref[...] = jnp.transpose(
        jnp.sum(x * Wv_ref[...], axis=1, keepdims=True) + bv_ref[...])


@functools.partial(jax.jit, static_argnames=())
def kernel(cent_obs, node_obs, adj, agent_id, W_embed, b_embed, Wg1, bg1,
           Wg2, bg2, gamma, beta, W1, b1, W2, b2, Wv, bv):
    bf16 = jnp.bfloat16
    nb = B // BB
    full = lambda shp: pl.BlockSpec(shp, lambda i: (0,) * len(shp))
    m2 = pl.pallas_call(
        _gnn_body,
        grid_spec=pl.GridSpec(
            grid=(nb,),
            in_specs=[
                pl.BlockSpec((BB, N, DNODE), lambda i: (i, 0, 0)),
                pl.BlockSpec((N, N, BB), lambda i: (0, 0, i)),
                pl.BlockSpec((1, BB), lambda i: (0, i)),
                full((DNODE, H)), full((1, H)),
                full((H, H)), full((1, H)),
            ],
            out_specs=pl.BlockSpec((BB, H), lambda i: (i, 0)),
        ),
        out_shape=jax.ShapeDtypeStruct((B, H), jnp.float32),
        compiler_params=pltpu.CompilerParams(
            dimension_semantics=("arbitrary",)),
    )(node_obs, jnp.transpose(adj, (1, 2, 0)),
      jnp.transpose(agent_id.astype(jnp.int32)),
      W_embed, b_embed.reshape(1, H),
      Wg1, bg1.reshape(1, H))

    full1 = lambda shp: pl.BlockSpec(shp, lambda: (0,) * len(shp))
    out = pl.pallas_call(
        _head_body,
        grid_spec=pl.GridSpec(
            grid=(),
            in_specs=[
                full1((B, H)), full1((B, DCENT)),
                full1((H, H)), full1((1, H)),
                full1((1, MLP_IN)), full1((1, MLP_IN)),
                full1((MLP_IN, H)), full1((1, H)),
                full1((H, H)), full1((1, H)),
                full1((1, H)), full1((1, 1)),
            ],
            out_specs=full1((1, B)),
        ),
        out_shape=jax.ShapeDtypeStruct((1, B), jnp.float32),
    )(m2, cent_obs,
      Wg2, bg2.reshape(1, H),
      gamma.reshape(1, MLP_IN), beta.reshape(1, MLP_IN),
      W1, b1.reshape(1, H),
      W2, b2.reshape(1, H),
      jnp.transpose(Wv), bv.reshape(1, 1))
    return jnp.transpose(out)


# speedup vs baseline: 1.1943x; 1.0007x over previous
import functools

import jax
import jax.numpy as jnp
from jax.experimental import pallas as pl
from jax.experimental.pallas import tpu as pltpu

B, N, DNODE, DCENT, H = 1024, 64, 128, 128, 256
MLP_IN = DCENT + H
BB = 128


def _gnn_body(node_ref, adj_ref, aid_ref, We_ref, be_ref,
              Wg1_ref, bg1_ref, m2_ref):
    f32 = jnp.float32
    bf16 = jnp.bfloat16
    X = node_ref[...].reshape(BB * N, DNODE).astype(bf16)
    h0 = jnp.maximum(
        jnp.dot(X, We_ref[...].astype(bf16), preferred_element_type=f32)
        + be_ref[...], 0.0).astype(bf16)
    adjt = adj_ref[...]
    adjb = jnp.transpose(adjt, (2, 0, 1))
    deg = jnp.maximum(jnp.sum(adjb, axis=2, keepdims=True), 1e-6)
    A = (adjb * (1.0 / deg)).astype(bf16)
    g = jnp.dot(h0, Wg1_ref[...].astype(bf16),
                preferred_element_type=f32)
    g3 = g.astype(bf16).reshape(BB, N, H)
    m = jax.lax.dot_general(A, g3,
                            (((2,), (1,)), ((0,), (0,))),
                            preferred_element_type=f32)
    h1 = jnp.maximum(m + bg1_ref[...], 0.0)
    aid2 = jnp.transpose(aid_ref[...])
    nidx = jax.lax.broadcasted_iota(jnp.int32, (BB, N), 1)
    onehotf = (nidx == aid2).astype(bf16)
    arow = jax.lax.dot_general(onehotf, A, (((1,), (1,)), ((0,), (0,))),
                               preferred_element_type=f32)
    m2_ref[...] = jax.lax.dot_general(
        arow, h1, (((1,), (1,)), ((0,), (0,))),
        preferred_element_type=f32)


def _head_body(m2_ref, cent_ref, Wg2_ref, bg2_ref, gam_ref, bet_ref,
               W1_ref, b1_ref, W2_ref, b2_ref, Wv_ref, bv_ref, out_ref):
    f32 = jnp.float32
    bf16 = jnp.bfloat16
    feats = jnp.maximum(
        jnp.dot(m2_ref[...].astype(bf16), Wg2_ref[...].astype(bf16),
                preferred_element_type=f32) + bg2_ref[...], 0.0)
    inp = jnp.concatenate([cent_ref[...], feats], axis=1)
    mu = jnp.mean(inp, axis=1, keepdims=True)
    var = jnp.mean(inp * inp, axis=1, keepdims=True) - mu * mu
    x = (inp - mu) * jax.lax.rsqrt(var + 1e-5) * gam_ref[...] + bet_ref[...]
    x = jnp.maximum(jnp.dot(x.astype(bf16), W1_ref[...].astype(bf16),
                            preferred_element_type=f32) + b1_ref[...], 0.0)
    x = jnp.maximum(jnp.dot(x.astype(bf16), W2_ref[...].astype(bf16),
                            preferred_element_type=f32) + b2_ref[...], 0.0)
    out_ref[...] = jnp.transpose(
        jnp.sum(x * Wv_ref[...], axis=1, keepdims=True) + bv_ref[...])


@functools.partial(jax.jit, static_argnames=())
def kernel(cent_obs, node_obs, adj, agent_id, W_embed, b_embed, Wg1, bg1,
           Wg2, bg2, gamma, beta, W1, b1, W2, b2, Wv, bv):
    nb = B // BB
    full = lambda shp: pl.BlockSpec(shp, lambda i: (0,) * len(shp))
    m2 = pl.pallas_call(
        _gnn_body,
        grid_spec=pl.GridSpec(
            grid=(nb,),
            in_specs=[
                pl.BlockSpec((BB, N, DNODE), lambda i: (i, 0, 0)),
                pl.BlockSpec((N, N, BB), lambda i: (0, 0, i)),
                pl.BlockSpec((1, BB), lambda i: (0, i)),
                full((DNODE, H)), full((1, H)),
                full((H, H)), full((1, H)),
            ],
            out_specs=pl.BlockSpec((BB, H), lambda i: (i, 0)),
        ),
        out_shape=jax.ShapeDtypeStruct((B, H), jnp.float32),
        compiler_params=pltpu.CompilerParams(
            dimension_semantics=("arbitrary",)),
    )(node_obs, jnp.transpose(adj, (1, 2, 0)),
      jnp.transpose(agent_id.astype(jnp.int32)),
      W_embed, b_embed.reshape(1, H),
      Wg1, bg1.reshape(1, H))

    full1 = lambda shp: pl.BlockSpec(shp, lambda: (0,) * len(shp))
    out = pl.pallas_call(
        _head_body,
        grid_spec=pl.GridSpec(
            grid=(),
            in_specs=[
                full1((B, H)), full1((B, DCENT)),
                full1((H, H)), full1((1, H)),
                full1((1, MLP_IN)), full1((1, MLP_IN)),
                full1((MLP_IN, H)), full1((1, H)),
                full1((H, H)), full1((1, H)),
                full1((1, H)), full1((1, 1)),
            ],
            out_specs=full1((1, B)),
        ),
        out_shape=jax.ShapeDtypeStruct((1, B), jnp.float32),
    )(m2, cent_obs,
      Wg2, bg2.reshape(1, H),
      gamma.reshape(1, MLP_IN), beta.reshape(1, MLP_IN),
      W1, b1.reshape(1, H),
      W2, b2.reshape(1, H),
      jnp.transpose(Wv), bv.reshape(1, 1))
    return jnp.transpose(out)
